# baseline (speedup 1.0000x reference)
import functools

import jax
import jax.numpy as jnp
from jax import lax
from jax.experimental import pallas as pl
from jax.experimental.pallas import tpu as pltpu
from jax.experimental.pallas import tpu_sc as plsc
from jax.experimental.layout import Format, Layout, with_layout_constraint

NUM_EMBEDDINGS = 1000000
DIM = 64
BATCH = 16384
SEQ = 50
SEQ_PAD = 56
DIM_PAD = 128
NUM_WORKERS = 32
ROWS_PER_WORKER = BATCH // NUM_WORKERS
SLAB = 8
NUM_SLABS = ROWS_PER_WORKER // SLAB

_mesh = plsc.VectorSubcoreMesh(core_axis_name="c", subcore_axis_name="s")


@functools.partial(
    pl.kernel,
    mesh=_mesh,
    out_type=jax.ShapeDtypeStruct((BATCH, SEQ_PAD, DIM_PAD), jnp.float32),
    scratch_types=[
        pltpu.VMEM((ROWS_PER_WORKER, SEQ), jnp.int32),
        pltpu.VMEM((SLAB, SEQ, DIM), jnp.float32),
        pltpu.VMEM((SLAB, SEQ, DIM), jnp.float32),
        pltpu.SemaphoreType.DMA,
        pltpu.SemaphoreType.DMA,
        pltpu.SemaphoreType.DMA,
        pltpu.SemaphoreType.DMA,
    ],
    compiler_params=pltpu.CompilerParams(use_tc_tiling_on_sc=False),
)
def _gather_kernel(table_hbm, idx_hbm, out_hbm,
                   idx_v, rows0, rows1, sg0, sg1, ss0, ss1):
    wid = lax.axis_index("s") * 2 + lax.axis_index("c")
    base = wid * ROWS_PER_WORKER
    pltpu.sync_copy(idx_hbm.at[pl.ds(base, ROWS_PER_WORKER)], idx_v)

    def start_gathers(k, buf, sem):
        for b in range(SLAB):
            pltpu.async_copy(
                table_hbm.at[idx_v.at[k * SLAB + b]], buf.at[b], sem)

    def wait_gathers(buf, sem):
        for b in range(SLAB):
            pltpu.make_async_copy(
                table_hbm.at[idx_v.at[0]], buf.at[b], sem).wait()

    def out_window(k):
        return out_hbm.at[pl.ds(base + k * SLAB, SLAB),
                          pl.ds(0, SEQ), pl.ds(0, DIM)]

    def start_store(k, buf, sem):
        pltpu.async_copy(buf, out_window(k), sem)

    def wait_store(buf, sem):
        pltpu.make_async_copy(buf, out_window(0), sem).wait()

    start_gathers(0, rows0, sg0)
    start_gathers(1, rows1, sg1)

    def body(g, carry):
        k = 2 * g
        wait_gathers(rows0, sg0)
        start_store(k, rows0, ss0)
        wait_gathers(rows1, sg1)
        start_store(k + 1, rows1, ss1)
        wait_store(rows0, ss0)
        start_gathers(k + 2, rows0, sg0)
        wait_store(rows1, ss1)
        start_gathers(k + 3, rows1, sg1)
        return carry

    lax.fori_loop(0, NUM_SLABS // 2 - 1, body, 0)

    tail = NUM_SLABS - 2
    wait_gathers(rows0, sg0)
    start_store(tail, rows0, ss0)
    wait_gathers(rows1, sg1)
    start_store(tail + 1, rows1, ss1)
    wait_store(rows0, ss0)
    wait_store(rows1, ss1)


def kernel(x, table):
    table = with_layout_constraint(table, Layout((1, 0)))
    table = lax.optimization_barrier(table)
    out = _gather_kernel(table, x.astype(jnp.int32))
    return out[:, :SEQ, :DIM]


# speedup vs baseline: 1.0143x; 1.0143x over previous
import functools

import jax
import jax.numpy as jnp
from jax import lax
from jax.experimental import pallas as pl
from jax.experimental.pallas import tpu as pltpu
from jax.experimental.pallas import tpu_sc as plsc

NUM_EMBEDDINGS = 1000000
DIM = 64
BATCH = 16384
SEQ = 50
SEQ_PAD = 56
DIM_PAD = 128
NUM_WORKERS = 32
ROWS_PER_WORKER = BATCH // NUM_WORKERS
SLAB = 8
NUM_SLABS = ROWS_PER_WORKER // SLAB

_mesh = plsc.VectorSubcoreMesh(core_axis_name="c", subcore_axis_name="s")


@functools.partial(
    pl.kernel,
    mesh=_mesh,
    out_type=jax.ShapeDtypeStruct((BATCH, SEQ_PAD, DIM_PAD), jnp.float32),
    scratch_types=[
        pltpu.VMEM((ROWS_PER_WORKER, SEQ), jnp.int32),
        pltpu.VMEM((SLAB, SEQ, DIM), jnp.float32),
        pltpu.VMEM((SLAB, SEQ, DIM), jnp.float32),
        pltpu.VMEM((SLAB, SEQ, DIM), jnp.float32),
        pltpu.SemaphoreType.DMA,
        pltpu.SemaphoreType.DMA,
        pltpu.SemaphoreType.DMA,
        pltpu.SemaphoreType.DMA,
        pltpu.SemaphoreType.DMA,
        pltpu.SemaphoreType.DMA,
    ],
    compiler_params=pltpu.CompilerParams(use_tc_tiling_on_sc=False),
)
def _gather_kernel(table_hbm, idx_hbm, out_hbm,
                   idx_v, rows0, rows1, rows2,
                   sg0, sg1, sg2, ss0, ss1, ss2):
    wid = lax.axis_index("s") * 2 + lax.axis_index("c")
    base = wid * ROWS_PER_WORKER
    pltpu.sync_copy(idx_hbm.at[pl.ds(base, ROWS_PER_WORKER)], idx_v)

    def start_gathers(k, buf, sem):
        for b in range(SLAB):
            pltpu.async_copy(
                table_hbm.at[idx_v.at[k * SLAB + b]], buf.at[b], sem)

    def wait_gathers(buf, sem):
        for b in range(SLAB):
            pltpu.make_async_copy(
                table_hbm.at[idx_v.at[0]], buf.at[b], sem).wait()

    def out_window(k):
        return out_hbm.at[pl.ds(base + k * SLAB, SLAB),
                          pl.ds(0, SEQ), pl.ds(0, DIM)]

    def start_store(k, buf, sem):
        pltpu.async_copy(buf, out_window(k), sem)

    def wait_store(buf, sem):
        pltpu.make_async_copy(buf, out_window(0), sem).wait()

    bufs = ((rows0, sg0, ss0), (rows1, sg1, ss1), (rows2, sg2, ss2))

    start_gathers(0, rows0, sg0)
    start_gathers(1, rows1, sg1)
    wait_gathers(rows0, sg0)
    start_store(0, rows0, ss0)
    start_gathers(2, rows2, sg2)

    def body(g, carry):
        for j in range(3):
            k = 3 * g + 1 + j
            buf, sg, ss = bufs[(1 + j) % 3]
            nbuf, nsg, nss = bufs[j]
            wait_gathers(buf, sg)
            start_store(k, buf, ss)
            wait_store(nbuf, nss)
            start_gathers(k + 2, nbuf, nsg)
        return carry

    lax.fori_loop(0, (NUM_SLABS - 4) // 3, body, 0)

    wait_store(rows0, ss0)
    start_gathers(NUM_SLABS - 1, rows0, sg0)
    for k in (NUM_SLABS - 3, NUM_SLABS - 2, NUM_SLABS - 1):
        buf, sg, ss = bufs[k % 3]
        wait_gathers(buf, sg)
        start_store(k, buf, ss)
    for buf, _, ss in bufs:
        wait_store(buf, ss)


def kernel(x, table):
    out = _gather_kernel(table, x.astype(jnp.int32))
    return out[:, :SEQ, :DIM]
